# Initial kernel scaffold; baseline (speedup 1.0000x reference)
import jax
import jax.numpy as jnp
from jax.experimental import pallas as pl


def kernel(inputs, router_W, router_b, integ_W, integ_b, codebooks):
    raise NotImplementedError("write your pallas kernel here")


# speedup vs baseline: 1.0714x; 1.0714x over previous
import functools

import jax
import jax.numpy as jnp
from jax import lax
from jax.experimental import pallas as pl
from jax.experimental.pallas import tpu as pltpu
from jax.experimental.pallas import tpu_sc as plsc


def _route_body(x_ref, rw_ref, rb_ref, cb_ref, w_ref, gidx_ref):
    x = x_ref[...]
    logits = (
        jnp.dot(x, rw_ref[...], preferred_element_type=jnp.float32)
        + rb_ref[...]
    )
    m = jnp.max(logits, axis=1, keepdims=True)
    e = jnp.exp(logits - m)
    w_ref[...] = e / jnp.sum(e, axis=1, keepdims=True)

    x2 = jnp.sum(x * x, axis=1, keepdims=True)
    C, K, _ = cb_ref.shape
    T = x.shape[0]
    iota = lax.broadcasted_iota(jnp.int32, (T, K), 1)
    cols = []
    for i in range(C):
        cb = cb_ref[i]
        c2 = jnp.sum(cb * cb, axis=1, keepdims=True).reshape(1, K)
        m2 = lax.dot_general(
            x, cb, (((1,), (1,)), ((), ())),
            preferred_element_type=jnp.float32,
        )
        d = x2 - 2.0 * m2 + c2
        dmin = jnp.min(d, axis=1, keepdims=True)
        idx = jnp.min(
            jnp.where(d == dmin, iota, K), axis=1, keepdims=True
        )
        cols.append(idx + i * K)
    gidx_ref[...] = jnp.concatenate(cols, axis=1)


def _route_call(flat, router_W, router_b, codebooks):
    N, H = flat.shape
    C, K, _ = codebooks.shape
    T = 512
    grid = (N // T,)
    return pl.pallas_call(
        _route_body,
        grid=grid,
        in_specs=[
            pl.BlockSpec((T, H), lambda t: (t, 0)),
            pl.BlockSpec((H, C), lambda t: (0, 0)),
            pl.BlockSpec((1, C), lambda t: (0, 0)),
            pl.BlockSpec((C, K, H), lambda t: (0, 0, 0)),
        ],
        out_specs=[
            pl.BlockSpec((T, C), lambda t: (t, 0)),
            pl.BlockSpec((T, C), lambda t: (t, 0)),
        ],
        out_shape=[
            jax.ShapeDtypeStruct((N, C), jnp.float32),
            jax.ShapeDtypeStruct((N, C), jnp.int32),
        ],
    )(flat, router_W, router_b.reshape(1, C), codebooks)


def _proj_body(cb_ref, w_ref, b_ref, p_ref):
    p_ref[...] = (
        jnp.dot(cb_ref[...], w_ref[...], preferred_element_type=jnp.float32)
        + b_ref[...]
    )


def _proj_call(cb_flat, integ_W, integ_b):
    CK, H = cb_flat.shape
    R = 512
    return pl.pallas_call(
        _proj_body,
        grid=(CK // R,),
        in_specs=[
            pl.BlockSpec((R, H), lambda r: (r, 0)),
            pl.BlockSpec((H, H), lambda r: (0, 0)),
            pl.BlockSpec((1, H), lambda r: (0, 0)),
        ],
        out_specs=pl.BlockSpec((R, H), lambda r: (r, 0)),
        out_shape=jax.ShapeDtypeStruct((CK, H), jnp.float32),
    )(cb_flat, integ_W, integ_b.reshape(1, H))


def _sc_combine_call(P, gidx_flat, w_flat, N, H, C):
    info = plsc.get_sparse_core_info()
    NC, NS, L = info.num_cores, info.num_subcores, info.num_lanes
    NW = NC * NS
    TPW = N // NW
    G = 4
    GR = G * C
    NG = TPW // G
    mesh = plsc.VectorSubcoreMesh(core_axis_name="c", subcore_axis_name="s")

    @functools.partial(
        pl.kernel,
        mesh=mesh,
        out_type=jax.ShapeDtypeStruct((N, H), jnp.float32),
        scratch_types=[
            pltpu.VMEM((TPW * C,), jnp.float32),
            pltpu.VMEM((GR,), jnp.int32),
            pltpu.VMEM((GR, H), jnp.float32),
            pltpu.VMEM((G, H), jnp.float32),
            pltpu.SemaphoreType.DMA,
        ],
    )
    def sc_kernel(p_hbm, gidx_hbm, w_hbm, out_hbm, w_v, idx_v, rows_v, out_v,
                  sem):
        wid = lax.axis_index("s") * NC + lax.axis_index("c")
        base = wid * TPW
        pltpu.sync_copy(w_hbm.at[pl.ds(base * C, TPW * C)], w_v)

        def group(g, carry):
            tok = base + g * G
            pltpu.sync_copy(gidx_hbm.at[pl.ds(tok * C, GR)], idx_v)
            pltpu.async_copy(p_hbm.at[idx_v], rows_v, sem).wait()
            wchunk = w_v[pl.ds(g * GR, GR)]
            for t in range(G):
                wv = [
                    jnp.zeros((L,), jnp.float32) + wchunk[t * C + i]
                    for i in range(C)
                ]
                for c in range(H // L):
                    sl = pl.ds(c * L, L)
                    acc = wv[0] * rows_v[t * C, sl]
                    for i in range(1, C):
                        acc = acc + wv[i] * rows_v[t * C + i, sl]
                    out_v[t, sl] = acc
            pltpu.sync_copy(out_v, out_hbm.at[pl.ds(tok, G)])
            return carry

        lax.fori_loop(0, NG, group, 0)

    return sc_kernel(P, gidx_flat, w_flat)


def kernel(inputs, router_W, router_b, integ_W, integ_b, codebooks):
    B, S, H = inputs.shape
    C, K, _ = codebooks.shape
    N = B * S
    flat = inputs.reshape(N, H)

    w_flat, gidx = _route_call(flat, router_W, router_b, codebooks)
    P = _proj_call(codebooks.reshape(C * K, H), integ_W, integ_b)
    out_flat = _sc_combine_call(
        P, gidx.reshape(-1), w_flat.reshape(-1), N, H, C
    )
    return out_flat.reshape(B, S, H), w_flat.reshape(B, S, C)
